# Initial kernel scaffold; baseline (speedup 1.0000x reference)
import jax
import jax.numpy as jnp
from jax.experimental import pallas as pl


def kernel(mem, idx, val, query):
    raise NotImplementedError("write your pallas kernel here")


# speedup vs baseline: 1.4019x; 1.4019x over previous
import functools

import jax
import jax.numpy as jnp
import numpy as np
from jax import lax
from jax.experimental import pallas as pl
from jax.experimental.pallas import tpu as pltpu
from jax.experimental.pallas import tpu_sc as plsc

M = 65536
D = 216
B = 4096
TOPK = 4
MP = M + 8
TS = 512
NT = M // TS
GPT = 128
GSZ = TS // GPT
NG = NT * GPT
NCAND = TOPK * GSZ
NW = 32
ROWS_PW = M // NW
CAP = 256
QB = 256
DP = 256
CAPX = CAP + 16
SCALE = np.float32(1.0) / np.float32(np.sqrt(np.float32(D)))
NEG = np.float32(-np.inf)

_INTERPRET = False


def _winner_body(idx_ref, widx_ref):
    x = idx_ref[...]
    def body(j, last):
        jrow = idx_ref[pl.ds(j, 1), :]
        eq = x[:, :, None] == jrow.reshape(1, 1, 128)
        jval = j * 128 + lax.broadcasted_iota(jnp.int32, (32, 128, 128), 2)
        cand = jnp.where(eq, jval, -1)
        return jnp.maximum(last, jnp.max(cand, axis=2))
    last = lax.fori_loop(0, 32, body, jnp.full((32, 128), -1, jnp.int32))
    biota = (lax.broadcasted_iota(jnp.int32, (32, 128), 0) * 128
             + lax.broadcasted_iota(jnp.int32, (32, 128), 1))
    widx_ref[...] = jnp.where(last == biota, x, M)


def _k1(idx32):
    return pl.pallas_call(
        _winner_body,
        out_shape=jax.ShapeDtypeStruct((32, 128), jnp.int32),
        interpret=_INTERPRET,
    )(idx32)


def _scatter_body(mem_hbm, widx_hbm, val_hbm, out_hbm,
                  widx_v, lidx_v, lval_v, gbuf, sem):
    wid = lax.axis_index("s") * 2 + lax.axis_index("c")
    lo = wid * ROWS_PW
    pltpu.sync_copy(mem_hbm.at[pl.ds(lo, ROWS_PW), :],
                    out_hbm.at[pl.ds(lo, ROWS_PW), :])
    pltpu.sync_copy(widx_hbm, widx_v)
    lanes = lax.broadcasted_iota(jnp.int32, (16,), 0)
    for c in range(CAPX // 16):
        lidx_v[pl.ds(c * 16, 16)] = jnp.full((16,), M, jnp.int32)
        lval_v[pl.ds(c * 16, 16)] = jnp.zeros((16,), jnp.int32)

    def body(c, cur):
        v = widx_v[pl.ds(c * 16, 16)]
        m = (v >= lo) & (v < lo + ROWS_PW)
        cs = plsc.cumsum(jnp.where(m, 1, 0))
        tgt = cur + cs - 1
        ok = m & (tgt < CAP)
        tgt = jnp.where(ok, tgt, CAP + lanes)
        plsc.store_scatter(lidx_v, [tgt], jnp.where(ok, v, M))
        plsc.store_scatter(lval_v, [tgt], jnp.where(ok, c * 16 + lanes, 0))
        return cur + jnp.max(cs)
    lax.fori_loop(0, B // 16, body, jnp.int32(0))
    pltpu.async_copy(val_hbm.at[lval_v], gbuf, sem).wait()
    pltpu.async_copy(gbuf, out_hbm.at[lidx_v], sem).wait()


def _k2(mem, widx, val):
    mesh = plsc.VectorSubcoreMesh(core_axis_name="c", subcore_axis_name="s",
                                  num_cores=2, num_subcores=16)
    f = pl.kernel(
        _scatter_body,
        out_type=jax.ShapeDtypeStruct((MP, D), jnp.float32),
        mesh=mesh,
        scratch_types=[
            pltpu.VMEM((B,), jnp.int32),
            pltpu.VMEM((CAPX,), jnp.int32),
            pltpu.VMEM((CAPX,), jnp.int32),
            pltpu.VMEM((CAPX, D), jnp.float32),
            pltpu.SemaphoreType.DMA,
        ],
        compiler_params=pltpu.CompilerParams(use_tc_tiling_on_sc=False, needs_layout_passes=False),
        interpret=_INTERPRET,
    )
    return f(mem, widx, val)


def _mm_body(q_ref, mem_ref, gm_ref):
    q = q_ref[...]
    mt = mem_ref[...]
    s = lax.dot_general(q, mt, (((1,), (1,)), ((), ())),
                        preferred_element_type=jnp.float32)
    gm_ref[...] = jnp.maximum(
        jnp.maximum(s[:, 0:128], s[:, 128:256]),
        jnp.maximum(s[:, 256:384], s[:, 384:512]))


def _k3(query, mem_new):
    return pl.pallas_call(
        _mm_body,
        grid=(NT,),
        in_specs=[
            pl.BlockSpec((B, D), lambda t: (0, 0)),
            pl.BlockSpec((TS, D), lambda t: (t, 0)),
        ],
        out_specs=pl.BlockSpec((B, GPT), lambda t: (0, t)),
        out_shape=jax.ShapeDtypeStruct((B, NG), jnp.float32),
        compiler_params=pltpu.CompilerParams(
            dimension_semantics=("arbitrary",)),
        interpret=_INTERPRET,
    )(query, mem_new)


def _sel_body(gm_ref, cid_ref):
    x = gm_ref[...]
    li = lax.broadcasted_iota(jnp.int32, (128, NG), 1)
    bases = []
    for _ in range(TOPK):
        m = jnp.max(x, axis=1, keepdims=True)
        p = jnp.min(jnp.where(x == m, li, NG), axis=1, keepdims=True)
        x = jnp.where(li == p, NEG, x)
        bases.append((p // GPT) * TS + (p % GPT))
    bs = jnp.concatenate(bases, axis=1)
    rows = bs[:, :, None] + GPT * lax.broadcasted_iota(
        jnp.int32, (128, TOPK, GSZ), 2)
    cid_ref[...] = rows.reshape(128, NCAND)


def _k4(gm):
    return pl.pallas_call(
        _sel_body,
        grid=(B // 128,),
        in_specs=[pl.BlockSpec((128, NG), lambda r: (r, 0))],
        out_specs=pl.BlockSpec((128, NCAND), lambda r: (r, 0)),
        out_shape=jax.ShapeDtypeStruct((B, NCAND), jnp.int32),
        compiler_params=pltpu.CompilerParams(
            dimension_semantics=("arbitrary",)),
        interpret=_INTERPRET,
    )(gm)


def _gather_body(mem_hbm, cid_hbm, out_hbm, cidx_v, gbuf, sem):
    wid = lax.axis_index("s") * 2 + lax.axis_index("c")
    base = wid * (B * NCAND // NW)
    for c in range((B * NCAND // NW) // 256):
        pltpu.sync_copy(cid_hbm.at[pl.ds(base + c * 256, 256)], cidx_v)
        pltpu.async_copy(mem_hbm.at[cidx_v], gbuf, sem).wait()
        pltpu.sync_copy(gbuf, out_hbm.at[pl.ds(base + c * 256, 256), :])


def _k5(mem_new, cids):
    mesh = plsc.VectorSubcoreMesh(core_axis_name="c", subcore_axis_name="s",
                                  num_cores=2, num_subcores=16)
    f = pl.kernel(
        _gather_body,
        out_type=jax.ShapeDtypeStruct((B * NCAND, D), jnp.float32),
        mesh=mesh,
        scratch_types=[
            pltpu.VMEM((256,), jnp.int32),
            pltpu.VMEM((256, D), jnp.float32),
            pltpu.SemaphoreType.DMA,
        ],
        compiler_params=pltpu.CompilerParams(use_tc_tiling_on_sc=False, needs_layout_passes=False),
        interpret=_INTERPRET,
    )
    return f(mem_new, cids)


def _combine_body(q_ref, cand_ref, out_ref):
    q = q_ref[...]
    c3 = cand_ref[...].reshape(QB, NCAND, D)
    cols = [jnp.sum(c3[:, c, :] * q, axis=1, keepdims=True)
            for c in range(NCAND)]
    s = jnp.concatenate(cols, axis=1) * SCALE
    li = lax.broadcasted_iota(jnp.int32, (QB, NCAND), 1)
    vals, poss = [], []
    x = s
    for _ in range(TOPK):
        m = jnp.max(x, axis=1, keepdims=True)
        p = jnp.min(jnp.where(x == m, li, NCAND), axis=1, keepdims=True)
        x = jnp.where(li == p, NEG, x)
        vals.append(m)
        poss.append(p)
    es = [jnp.exp(v - vals[0]) for v in vals]
    den = ((es[0] + es[1]) + (es[2] + es[3]))
    wc = jnp.zeros((QB, NCAND), jnp.float32)
    for k in range(TOPK):
        wc = jnp.where(li == poss[k], es[k] / den, wc)
    acc = jnp.zeros((QB, D), jnp.float32)
    for c in range(NCAND):
        acc = acc + wc[:, c:c + 1] * c3[:, c, :]
    out_ref[...] = acc


def _k6(query, cand):
    return pl.pallas_call(
        _combine_body,
        grid=(B // QB,),
        in_specs=[
            pl.BlockSpec((QB, D), lambda i: (i, 0)),
            pl.BlockSpec((QB * NCAND, D), lambda i: (i, 0)),
        ],
        out_specs=pl.BlockSpec((QB, D), lambda i: (i, 0)),
        out_shape=jax.ShapeDtypeStruct((B, D), jnp.float32),
        compiler_params=pltpu.CompilerParams(
            dimension_semantics=("arbitrary",)),
        interpret=_INTERPRET,
    )(query, cand)


def kernel(mem, idx, val, query):
    widx = _k1(idx.reshape(32, 128)).reshape(B)
    mem_new = _k2(mem, widx, val)
    gm = _k3(query, mem_new)
    cids = _k4(gm).reshape(B * NCAND)
    cand = _k5(mem_new, cids)
    return _k6(query, cand)


# speedup vs baseline: 2.7215x; 1.9413x over previous
import functools

import jax
import jax.numpy as jnp
import numpy as np
from jax import lax
from jax.experimental import pallas as pl
from jax.experimental.pallas import tpu as pltpu
from jax.experimental.pallas import tpu_sc as plsc

M = 65536
D = 216
B = 4096
TOPK = 4
MP = M + 8
TS = 512
NT = M // TS
GPT = 128
GSZ = TS // GPT
NG = NT * GPT
NCAND = TOPK * GSZ
NW = 32
ROWS_PW = M // NW
CAP = 256
QB = 256
DP = 256
CAPX = CAP + 16
SCALE = np.float32(1.0) / np.float32(np.sqrt(np.float32(D)))
NEG = np.float32(-np.inf)

_INTERPRET = False


def _winner_body(idx_ref, widx_ref):
    x = idx_ref[...]
    def body(j, last):
        jrow = idx_ref[pl.ds(j, 1), :]
        eq = x[:, :, None] == jrow.reshape(1, 1, 128)
        jval = j * 128 + lax.broadcasted_iota(jnp.int32, (32, 128, 128), 2)
        cand = jnp.where(eq, jval, -1)
        return jnp.maximum(last, jnp.max(cand, axis=2))
    last = lax.fori_loop(0, 32, body, jnp.full((32, 128), -1, jnp.int32))
    biota = (lax.broadcasted_iota(jnp.int32, (32, 128), 0) * 128
             + lax.broadcasted_iota(jnp.int32, (32, 128), 1))
    widx_ref[...] = jnp.where(last == biota, x, M)


def _k1(idx32):
    return pl.pallas_call(
        _winner_body,
        out_shape=jax.ShapeDtypeStruct((32, 128), jnp.int32),
        interpret=_INTERPRET,
    )(idx32)


def _scatter_body(mem_hbm, widx_hbm, val_hbm, out_hbm,
                  buf, widx_v, lidx_v, lval_v, gbuf, sem):
    wid = lax.axis_index("s") * 2 + lax.axis_index("c")
    lo = wid * ROWS_PW
    for c in range(ROWS_PW // 128):
        pltpu.sync_copy(mem_hbm.at[pl.ds(lo + c * 128, 128), :], buf)
        pltpu.sync_copy(buf, out_hbm.at[pl.ds(lo + c * 128, 128), :])
    pltpu.sync_copy(widx_hbm, widx_v)
    lanes = lax.broadcasted_iota(jnp.int32, (16,), 0)
    for c in range(CAPX // 16):
        lidx_v[pl.ds(c * 16, 16)] = jnp.full((16,), M, jnp.int32)
        lval_v[pl.ds(c * 16, 16)] = jnp.zeros((16,), jnp.int32)

    def body(c, cur):
        v = widx_v[pl.ds(c * 16, 16)]
        m = (v >= lo) & (v < lo + ROWS_PW)
        cs = plsc.cumsum(jnp.where(m, 1, 0))
        tgt = cur + cs - 1
        ok = m & (tgt < CAP)
        tgt = jnp.where(ok, tgt, CAP + lanes)
        plsc.store_scatter(lidx_v, [tgt], jnp.where(ok, v, M))
        plsc.store_scatter(lval_v, [tgt], jnp.where(ok, c * 16 + lanes, 0))
        return cur + jnp.max(cs)
    lax.fori_loop(0, B // 16, body, jnp.int32(0))
    pltpu.async_copy(val_hbm.at[lval_v], gbuf, sem).wait()
    pltpu.async_copy(gbuf, out_hbm.at[lidx_v], sem).wait()


def _k2(mem, widx, val):
    mesh = plsc.VectorSubcoreMesh(core_axis_name="c", subcore_axis_name="s",
                                  num_cores=2, num_subcores=16)
    f = pl.kernel(
        _scatter_body,
        out_type=jax.ShapeDtypeStruct((MP, D), jnp.float32),
        mesh=mesh,
        scratch_types=[
            pltpu.VMEM((128, D), jnp.float32),
            pltpu.VMEM((B,), jnp.int32),
            pltpu.VMEM((CAPX,), jnp.int32),
            pltpu.VMEM((CAPX,), jnp.int32),
            pltpu.VMEM((CAPX, D), jnp.float32),
            pltpu.SemaphoreType.DMA,
        ],
        compiler_params=pltpu.CompilerParams(use_tc_tiling_on_sc=False, needs_layout_passes=False),
        interpret=_INTERPRET,
    )
    return f(mem, widx, val)


def _mm_body(q_ref, mem_ref, gm_ref):
    q = q_ref[...]
    mt = mem_ref[...]
    s = lax.dot_general(q, mt, (((1,), (1,)), ((), ())),
                        preferred_element_type=jnp.float32)
    gm_ref[...] = jnp.maximum(
        jnp.maximum(s[:, 0:128], s[:, 128:256]),
        jnp.maximum(s[:, 256:384], s[:, 384:512]))


def _k3(query, mem_new):
    return pl.pallas_call(
        _mm_body,
        grid=(NT,),
        in_specs=[
            pl.BlockSpec((B, D), lambda t: (0, 0)),
            pl.BlockSpec((TS, D), lambda t: (t, 0)),
        ],
        out_specs=pl.BlockSpec((B, GPT), lambda t: (0, t)),
        out_shape=jax.ShapeDtypeStruct((B, NG), jnp.float32),
        compiler_params=pltpu.CompilerParams(
            dimension_semantics=("arbitrary",)),
        interpret=_INTERPRET,
    )(query, mem_new)


def _sel_body(gm_ref, cid_ref):
    x = gm_ref[...]
    li = lax.broadcasted_iota(jnp.int32, (128, NG), 1)
    bases = []
    for _ in range(TOPK):
        m = jnp.max(x, axis=1, keepdims=True)
        p = jnp.min(jnp.where(x == m, li, NG), axis=1, keepdims=True)
        x = jnp.where(li == p, NEG, x)
        bases.append((p // GPT) * TS + (p % GPT))
    bs = jnp.concatenate(bases, axis=1)
    rows = bs[:, :, None] + GPT * lax.broadcasted_iota(
        jnp.int32, (128, TOPK, GSZ), 2)
    cid_ref[...] = rows.reshape(128, NCAND)


def _k4(gm):
    return pl.pallas_call(
        _sel_body,
        grid=(B // 128,),
        in_specs=[pl.BlockSpec((128, NG), lambda r: (r, 0))],
        out_specs=pl.BlockSpec((128, NCAND), lambda r: (r, 0)),
        out_shape=jax.ShapeDtypeStruct((B, NCAND), jnp.int32),
        compiler_params=pltpu.CompilerParams(
            dimension_semantics=("arbitrary",)),
        interpret=_INTERPRET,
    )(gm)


def _gather_body(mem_hbm, cid_hbm, out_hbm, cidx_v, gbuf, sem):
    wid = lax.axis_index("s") * 2 + lax.axis_index("c")
    base = wid * (B * NCAND // NW)
    for c in range((B * NCAND // NW) // 256):
        pltpu.sync_copy(cid_hbm.at[pl.ds(base + c * 256, 256)], cidx_v)
        pltpu.async_copy(mem_hbm.at[cidx_v], gbuf, sem).wait()
        pltpu.sync_copy(gbuf, out_hbm.at[pl.ds(base + c * 256, 256), :])


def _k5(mem_new, cids):
    mesh = plsc.VectorSubcoreMesh(core_axis_name="c", subcore_axis_name="s",
                                  num_cores=2, num_subcores=16)
    f = pl.kernel(
        _gather_body,
        out_type=jax.ShapeDtypeStruct((B * NCAND, D), jnp.float32),
        mesh=mesh,
        scratch_types=[
            pltpu.VMEM((256,), jnp.int32),
            pltpu.VMEM((256, D), jnp.float32),
            pltpu.SemaphoreType.DMA,
        ],
        compiler_params=pltpu.CompilerParams(use_tc_tiling_on_sc=False, needs_layout_passes=False),
        interpret=_INTERPRET,
    )
    return f(mem_new, cids)


def _combine_body(q_ref, cand_ref, out_ref):
    q = q_ref[...]
    c3 = cand_ref[...].reshape(QB, NCAND, D)
    cols = [jnp.sum(c3[:, c, :] * q, axis=1, keepdims=True)
            for c in range(NCAND)]
    s = jnp.concatenate(cols, axis=1) * SCALE
    li = lax.broadcasted_iota(jnp.int32, (QB, NCAND), 1)
    vals, poss = [], []
    x = s
    for _ in range(TOPK):
        m = jnp.max(x, axis=1, keepdims=True)
        p = jnp.min(jnp.where(x == m, li, NCAND), axis=1, keepdims=True)
        x = jnp.where(li == p, NEG, x)
        vals.append(m)
        poss.append(p)
    es = [jnp.exp(v - vals[0]) for v in vals]
    den = ((es[0] + es[1]) + (es[2] + es[3]))
    wc = jnp.zeros((QB, NCAND), jnp.float32)
    for k in range(TOPK):
        wc = jnp.where(li == poss[k], es[k] / den, wc)
    acc = jnp.zeros((QB, D), jnp.float32)
    for c in range(NCAND):
        acc = acc + wc[:, c:c + 1] * c3[:, c, :]
    out_ref[...] = acc


def _k6(query, cand):
    return pl.pallas_call(
        _combine_body,
        grid=(B // QB,),
        in_specs=[
            pl.BlockSpec((QB, D), lambda i: (i, 0)),
            pl.BlockSpec((QB * NCAND, D), lambda i: (i, 0)),
        ],
        out_specs=pl.BlockSpec((QB, D), lambda i: (i, 0)),
        out_shape=jax.ShapeDtypeStruct((B, D), jnp.float32),
        compiler_params=pltpu.CompilerParams(
            dimension_semantics=("arbitrary",)),
        interpret=_INTERPRET,
    )(query, cand)


def kernel(mem, idx, val, query):
    widx = _k1(idx.reshape(32, 128)).reshape(B)
    mem_new = _k2(mem, widx, val)
    gm = _k3(query, mem_new)
    cids = _k4(gm).reshape(B * NCAND)
    cand = _k5(mem_new, cids)
    return _k6(query, cand)


# speedup vs baseline: 2.7782x; 1.0208x over previous
import functools

import jax
import jax.numpy as jnp
import numpy as np
from jax import lax
from jax.experimental import pallas as pl
from jax.experimental.pallas import tpu as pltpu
from jax.experimental.pallas import tpu_sc as plsc

M = 65536
D = 216
B = 4096
TOPK = 4
MP = M + 8
TS = 512
NT = M // TS
GPT = 128
GSZ = TS // GPT
NG = NT * GPT
NCAND = TOPK * GSZ
NW = 32
ROWS_PW = M // NW
CAP = 256
QB = 256
DP = 256
CAPX = CAP + 16
SCALE = np.float32(1.0) / np.float32(np.sqrt(np.float32(D)))
NEG = np.float32(-np.inf)

_INTERPRET = False


def _winner_body(idx_ref, widx_ref):
    x = idx_ref[...]
    def body(j, last):
        jrow = idx_ref[pl.ds(j, 1), :]
        eq = x[:, :, None] == jrow.reshape(1, 1, 128)
        jval = j * 128 + lax.broadcasted_iota(jnp.int32, (32, 128, 128), 2)
        cand = jnp.where(eq, jval, -1)
        return jnp.maximum(last, jnp.max(cand, axis=2))
    last = lax.fori_loop(0, 32, body, jnp.full((32, 128), -1, jnp.int32))
    biota = (lax.broadcasted_iota(jnp.int32, (32, 128), 0) * 128
             + lax.broadcasted_iota(jnp.int32, (32, 128), 1))
    widx_ref[...] = jnp.where(last == biota, x, M)


def _k1(idx32):
    return pl.pallas_call(
        _winner_body,
        out_shape=jax.ShapeDtypeStruct((32, 128), jnp.int32),
        interpret=_INTERPRET,
    )(idx32)


def _scatter_body(mem_hbm, widx_hbm, val_hbm, out_hbm,
                  buf, widx_v, lidx_v, lval_v, gbuf, sem,
                  isem0, isem1, osem0, osem1):
    wid = lax.axis_index("s") * 2 + lax.axis_index("c")
    lo = wid * ROWS_PW
    nc = ROWS_PW // 128
    isems = [isem0, isem1]
    osems = [osem0, osem1]
    in_cp = [None] * nc
    out_cp = [None] * nc
    in_cp[0] = pltpu.make_async_copy(
        mem_hbm.at[pl.ds(lo, 128), :], buf.at[0], isems[0])
    in_cp[0].start()
    for c in range(nc):
        if c + 1 < nc:
            if c >= 1:
                out_cp[c - 1].wait()
            in_cp[c + 1] = pltpu.make_async_copy(
                mem_hbm.at[pl.ds(lo + (c + 1) * 128, 128), :],
                buf.at[(c + 1) % 2], isems[(c + 1) % 2])
            in_cp[c + 1].start()
        in_cp[c].wait()
        out_cp[c] = pltpu.make_async_copy(
            buf.at[c % 2], out_hbm.at[pl.ds(lo + c * 128, 128), :],
            osems[c % 2])
        out_cp[c].start()
    out_cp[nc - 2].wait()
    out_cp[nc - 1].wait()
    pltpu.sync_copy(widx_hbm, widx_v)
    lanes = lax.broadcasted_iota(jnp.int32, (16,), 0)
    for c in range(CAPX // 16):
        lidx_v[pl.ds(c * 16, 16)] = jnp.full((16,), M, jnp.int32)
        lval_v[pl.ds(c * 16, 16)] = jnp.zeros((16,), jnp.int32)

    def body(c, cur):
        v = widx_v[pl.ds(c * 16, 16)]
        m = (v >= lo) & (v < lo + ROWS_PW)
        cs = plsc.cumsum(jnp.where(m, 1, 0))
        tgt = cur + cs - 1
        ok = m & (tgt < CAP)
        tgt = jnp.where(ok, tgt, CAP + lanes)
        plsc.store_scatter(lidx_v, [tgt], jnp.where(ok, v, M))
        plsc.store_scatter(lval_v, [tgt], jnp.where(ok, c * 16 + lanes, 0))
        return cur + jnp.max(cs)
    lax.fori_loop(0, B // 16, body, jnp.int32(0))
    pltpu.async_copy(val_hbm.at[lval_v], gbuf, sem).wait()
    pltpu.async_copy(gbuf, out_hbm.at[lidx_v], sem).wait()


def _k2(mem, widx, val):
    mesh = plsc.VectorSubcoreMesh(core_axis_name="c", subcore_axis_name="s",
                                  num_cores=2, num_subcores=16)
    f = pl.kernel(
        _scatter_body,
        out_type=jax.ShapeDtypeStruct((MP, D), jnp.float32),
        mesh=mesh,
        scratch_types=[
            pltpu.VMEM((2, 128, D), jnp.float32),
            pltpu.VMEM((B,), jnp.int32),
            pltpu.VMEM((CAPX,), jnp.int32),
            pltpu.VMEM((CAPX,), jnp.int32),
            pltpu.VMEM((CAPX, D), jnp.float32),
            pltpu.SemaphoreType.DMA,
            pltpu.SemaphoreType.DMA,
            pltpu.SemaphoreType.DMA,
            pltpu.SemaphoreType.DMA,
            pltpu.SemaphoreType.DMA,
        ],
        compiler_params=pltpu.CompilerParams(use_tc_tiling_on_sc=False, needs_layout_passes=False),
        interpret=_INTERPRET,
    )
    return f(mem, widx, val)


def _mm_body(q_ref, mem_ref, gm_ref):
    q = q_ref[...]
    mt = mem_ref[...]
    s = lax.dot_general(q, mt, (((1,), (1,)), ((), ())),
                        preferred_element_type=jnp.float32)
    gm_ref[...] = jnp.maximum(
        jnp.maximum(s[:, 0:128], s[:, 128:256]),
        jnp.maximum(s[:, 256:384], s[:, 384:512]))


def _k3(query, mem_new):
    return pl.pallas_call(
        _mm_body,
        grid=(NT,),
        in_specs=[
            pl.BlockSpec((B, D), lambda t: (0, 0)),
            pl.BlockSpec((TS, D), lambda t: (t, 0)),
        ],
        out_specs=pl.BlockSpec((B, GPT), lambda t: (0, t)),
        out_shape=jax.ShapeDtypeStruct((B, NG), jnp.float32),
        compiler_params=pltpu.CompilerParams(
            dimension_semantics=("arbitrary",)),
        interpret=_INTERPRET,
    )(query, mem_new)


def _sel_body(gm_ref, cid_ref):
    x = gm_ref[...]
    li = lax.broadcasted_iota(jnp.int32, (128, NG), 1)
    bases = []
    for _ in range(TOPK):
        m = jnp.max(x, axis=1, keepdims=True)
        p = jnp.min(jnp.where(x == m, li, NG), axis=1, keepdims=True)
        x = jnp.where(li == p, NEG, x)
        bases.append((p // GPT) * TS + (p % GPT))
    bs = jnp.concatenate(bases, axis=1)
    rows = bs[:, :, None] + GPT * lax.broadcasted_iota(
        jnp.int32, (128, TOPK, GSZ), 2)
    cid_ref[...] = rows.reshape(128, NCAND)


def _k4(gm):
    return pl.pallas_call(
        _sel_body,
        grid=(B // 128,),
        in_specs=[pl.BlockSpec((128, NG), lambda r: (r, 0))],
        out_specs=pl.BlockSpec((128, NCAND), lambda r: (r, 0)),
        out_shape=jax.ShapeDtypeStruct((B, NCAND), jnp.int32),
        compiler_params=pltpu.CompilerParams(
            dimension_semantics=("arbitrary",)),
        interpret=_INTERPRET,
    )(gm)


def _gather_body(mem_hbm, cid_hbm, out_hbm, cidx_v, gbuf, sem):
    wid = lax.axis_index("s") * 2 + lax.axis_index("c")
    base = wid * (B * NCAND // NW)
    for c in range((B * NCAND // NW) // 256):
        pltpu.sync_copy(cid_hbm.at[pl.ds(base + c * 256, 256)], cidx_v)
        pltpu.async_copy(mem_hbm.at[cidx_v], gbuf, sem).wait()
        pltpu.sync_copy(gbuf, out_hbm.at[pl.ds(base + c * 256, 256), :])


def _k5(mem_new, cids):
    mesh = plsc.VectorSubcoreMesh(core_axis_name="c", subcore_axis_name="s",
                                  num_cores=2, num_subcores=16)
    f = pl.kernel(
        _gather_body,
        out_type=jax.ShapeDtypeStruct((B * NCAND, D), jnp.float32),
        mesh=mesh,
        scratch_types=[
            pltpu.VMEM((256,), jnp.int32),
            pltpu.VMEM((256, D), jnp.float32),
            pltpu.SemaphoreType.DMA,
        ],
        compiler_params=pltpu.CompilerParams(use_tc_tiling_on_sc=False, needs_layout_passes=False),
        interpret=_INTERPRET,
    )
    return f(mem_new, cids)


def _combine_body(q_ref, cand_ref, out_ref):
    q = q_ref[...]
    c3 = cand_ref[...].reshape(QB, NCAND, D)
    cols = [jnp.sum(c3[:, c, :] * q, axis=1, keepdims=True)
            for c in range(NCAND)]
    s = jnp.concatenate(cols, axis=1) * SCALE
    li = lax.broadcasted_iota(jnp.int32, (QB, NCAND), 1)
    vals, poss = [], []
    x = s
    for _ in range(TOPK):
        m = jnp.max(x, axis=1, keepdims=True)
        p = jnp.min(jnp.where(x == m, li, NCAND), axis=1, keepdims=True)
        x = jnp.where(li == p, NEG, x)
        vals.append(m)
        poss.append(p)
    es = [jnp.exp(v - vals[0]) for v in vals]
    den = ((es[0] + es[1]) + (es[2] + es[3]))
    wc = jnp.zeros((QB, NCAND), jnp.float32)
    for k in range(TOPK):
        wc = jnp.where(li == poss[k], es[k] / den, wc)
    acc = jnp.zeros((QB, D), jnp.float32)
    for c in range(NCAND):
        acc = acc + wc[:, c:c + 1] * c3[:, c, :]
    out_ref[...] = acc


def _k6(query, cand):
    return pl.pallas_call(
        _combine_body,
        grid=(B // QB,),
        in_specs=[
            pl.BlockSpec((QB, D), lambda i: (i, 0)),
            pl.BlockSpec((QB * NCAND, D), lambda i: (i, 0)),
        ],
        out_specs=pl.BlockSpec((QB, D), lambda i: (i, 0)),
        out_shape=jax.ShapeDtypeStruct((B, D), jnp.float32),
        compiler_params=pltpu.CompilerParams(
            dimension_semantics=("arbitrary",)),
        interpret=_INTERPRET,
    )(query, cand)


def kernel(mem, idx, val, query):
    widx = _k1(idx.reshape(32, 128)).reshape(B)
    mem_new = _k2(mem, widx, val)
    gm = _k3(query, mem_new)
    cids = _k4(gm).reshape(B * NCAND)
    cand = _k5(mem_new, cids)
    return _k6(query, cand)
